# Initial kernel scaffold; baseline (speedup 1.0000x reference)
import jax
import jax.numpy as jnp
from jax.experimental import pallas as pl


def kernel(indices, weight):
    raise NotImplementedError("write your pallas kernel here")


# speedup vs baseline: 1.6820x; 1.6820x over previous
import functools

import jax
import jax.numpy as jnp
from jax import lax
from jax.experimental import pallas as pl
from jax.experimental.pallas import tpu as pltpu
from jax.experimental.pallas import tpu_sc as plsc

NC = 2
NS = 16
NW = NC * NS

D = 64
CH = 128


def _make_gather(batch):
    assert batch % (NW * CH) == 0
    nch = batch // (NW * CH)
    mesh = plsc.VectorSubcoreMesh(core_axis_name="c", subcore_axis_name="s")

    @functools.partial(
        pl.kernel,
        mesh=mesh,
        compiler_params=pltpu.CompilerParams(use_tc_tiling_on_sc=False),
        out_type=jax.ShapeDtypeStruct((NW, nch, CH, D), jnp.float32),
        scratch_types=[
            pltpu.VMEM((nch, CH), jnp.int32),
            pltpu.VMEM((CH, D), jnp.float32),
            pltpu.SemaphoreType.DMA,
        ],
    )
    def gather(idx_hbm, table_hbm, out_hbm, idx_v, rows_v, sem):
        wid = lax.axis_index("s") * NC + lax.axis_index("c")
        pltpu.sync_copy(idx_hbm.at[wid], idx_v)

        def step(c, carry):
            pltpu.async_copy(table_hbm.at[idx_v.at[c]], rows_v, sem).wait()
            pltpu.sync_copy(rows_v, out_hbm.at[wid, c])
            return carry

        lax.fori_loop(0, nch, step, 0)

    return gather


def kernel(indices, weight):
    batch, hist = indices.shape
    total = batch * hist
    idx = indices.reshape(NW, total // (NW * CH), CH).astype(jnp.int32)
    out = _make_gather(total)(idx, weight)
    return out.reshape(batch, hist, D)


# speedup vs baseline: 1.8769x; 1.1159x over previous
import functools

import jax
import jax.numpy as jnp
from jax import lax
from jax.experimental import pallas as pl
from jax.experimental.pallas import tpu as pltpu
from jax.experimental.pallas import tpu_sc as plsc

NC = 2
NS = 16
NW = NC * NS

D = 64
CH = 128


NBUF = 4


def _make_gather(batch):
    assert batch % (NW * CH * NBUF) == 0
    nch = batch // (NW * CH)
    ngrp = nch // NBUF
    mesh = plsc.VectorSubcoreMesh(core_axis_name="c", subcore_axis_name="s")

    @functools.partial(
        pl.kernel,
        mesh=mesh,
        compiler_params=pltpu.CompilerParams(use_tc_tiling_on_sc=False),
        out_type=jax.ShapeDtypeStruct((NW, nch, CH, D), jnp.float32),
        scratch_types=[
            pltpu.VMEM((nch, CH), jnp.int32),
            pltpu.VMEM((NBUF, CH, D), jnp.float32),
            pltpu.SemaphoreType.DMA((NBUF,)),
        ],
    )
    def gather(idx_hbm, table_hbm, out_hbm, idx_v, rows_v, sem):
        wid = lax.axis_index("s") * NC + lax.axis_index("c")
        pltpu.sync_copy(idx_hbm.at[wid], idx_v)

        for b in range(NBUF):
            pltpu.async_copy(table_hbm.at[idx_v.at[b]], rows_v.at[b], sem.at[b])

        def group(g, carry):
            for b in range(NBUF):
                c = g * NBUF + b
                pltpu.make_async_copy(
                    table_hbm.at[idx_v.at[c]], rows_v.at[b], sem.at[b]
                ).wait()
                pltpu.sync_copy(rows_v.at[b], out_hbm.at[wid, c])
                pltpu.async_copy(
                    table_hbm.at[idx_v.at[c + NBUF]], rows_v.at[b], sem.at[b]
                )
            return carry

        lax.fori_loop(0, ngrp - 1, group, 0)

        for b in range(NBUF):
            c = (ngrp - 1) * NBUF + b
            pltpu.make_async_copy(
                table_hbm.at[idx_v.at[c]], rows_v.at[b], sem.at[b]
            ).wait()
            pltpu.sync_copy(rows_v.at[b], out_hbm.at[wid, c])

    return gather


def kernel(indices, weight):
    batch, hist = indices.shape
    total = batch * hist
    idx = indices.reshape(NW, total // (NW * CH), CH).astype(jnp.int32)
    out = _make_gather(total)(idx, weight)
    return out.reshape(batch, hist, D)
